# baseline (speedup 1.0000x reference)
import jax
import jax.numpy as jnp
from jax import lax
from jax.experimental import pallas as pl
from jax.experimental.pallas import tpu as pltpu

_M, _K = 2, 3
_NNZ = 6


def _spmm_kernel(y_ref, coo_ref, out_ref):
    c = [[jnp.float32(0.0)] * _K for _ in range(_M)]
    for k in range(_NNZ):
        r = coo_ref[0, k]
        col = coo_ref[1, k]
        v = lax.bitcast_convert_type(coo_ref[2, k], jnp.float32)
        for i in range(_M):
            for j in range(_K):
                hit = jnp.logical_and(r == i, col == j)
                c[i][j] = c[i][j] + jnp.where(hit, v, jnp.float32(0.0))
    yb = y_ref[...]
    for i in range(_M):
        acc = c[i][0] * yb[0:1, :]
        for j in range(1, _K):
            acc = acc + c[i][j] * yb[j : j + 1, :]
        out_ref[i : i + 1, :] = acc


def kernel(y, xind, xval):
    coo = jnp.concatenate(
        [xind.astype(jnp.int32),
         lax.bitcast_convert_type(xval, jnp.int32)[None, :]])
    return pl.pallas_call(
        _spmm_kernel,
        out_shape=jax.ShapeDtypeStruct((_M, y.shape[1]), y.dtype),
        in_specs=[
            pl.BlockSpec(memory_space=pltpu.VMEM),
            pl.BlockSpec(memory_space=pltpu.SMEM),
        ],
        out_specs=pl.BlockSpec(memory_space=pltpu.VMEM),
    )(y, coo)
